# Initial kernel scaffold; baseline (speedup 1.0000x reference)
import jax
import jax.numpy as jnp
from jax.experimental import pallas as pl


def kernel(x_a, edge_index_a, edge_attr_a, batch_a, enz_a, x_b, edge_index_b, edge_attr_b, batch_b, enz_b, params):
    raise NotImplementedError("write your pallas kernel here")


# speedup vs baseline: 10.7436x; 10.7436x over previous
import functools

import jax
import jax.numpy as jnp
from jax import lax
from jax.experimental import pallas as pl
from jax.experimental.pallas import tpu as pltpu
from jax.experimental.pallas import tpu_sc as plsc

NW = 32
CH = 128
E_PAD = 163840


def _sc_gather(table, idx, D):
  B = idx.shape[0]
  bpw = B // NW
  nch = bpw // CH
  mesh = plsc.VectorSubcoreMesh(core_axis_name="c", subcore_axis_name="s")

  @functools.partial(
      pl.kernel, mesh=mesh,
      out_type=jax.ShapeDtypeStruct((B, D), jnp.float32),
      scratch_types=[
          pltpu.VMEM((CH,), jnp.int32),
          pltpu.VMEM((CH, D), jnp.float32),
          pltpu.SemaphoreType.DMA,
      ],
  )
  def k(table_hbm, idx_hbm, out_hbm, idx_v, rows_v, sem):
    wid = lax.axis_index("s") * 2 + lax.axis_index("c")

    def body(i, carry):
      base = wid * bpw + i * CH
      pltpu.sync_copy(idx_hbm.at[pl.ds(base, CH)], idx_v)
      pltpu.async_copy(table_hbm.at[idx_v], rows_v, sem).wait()
      pltpu.sync_copy(rows_v, out_hbm.at[pl.ds(base, CH)])
      return carry

    lax.fori_loop(0, nch, body, 0)

  return k(table, idx)


def _sc_scatter_add(vals, idx, zeros_nd, N, D):
  B = idx.shape[0]
  bpw = B // NW
  nch = bpw // CH
  mesh = plsc.VectorSubcoreMesh(core_axis_name="c", subcore_axis_name="s")

  @functools.partial(
      pl.kernel, mesh=mesh,
      out_type=jax.ShapeDtypeStruct((2, N, D), jnp.float32),
      scratch_types=[
          pltpu.VMEM((CH,), jnp.int32),
          pltpu.VMEM((CH, D), jnp.float32),
          pltpu.VMEM_SHARED((N, D), jnp.float32),
          pltpu.SemaphoreType.DMA,
      ],
  )
  def k(vals_hbm, idx_hbm, zero_hbm, out_hbm, idx_v, vals_v, table_sh, sem):
    cid = lax.axis_index("c")
    sid = lax.axis_index("s")
    wid = sid * 2 + cid

    @pl.when(sid == 0)
    def _init():
      pltpu.sync_copy(zero_hbm, table_sh)

    plsc.subcore_barrier()

    def body(i, carry):
      base = wid * bpw + i * CH
      pltpu.sync_copy(idx_hbm.at[pl.ds(base, CH)], idx_v)
      pltpu.sync_copy(vals_hbm.at[pl.ds(base, CH)], vals_v)
      pltpu.sync_copy(vals_v, table_sh.at[idx_v], add=True)
      return carry

    lax.fori_loop(0, nch, body, 0)
    plsc.subcore_barrier()

    @pl.when(sid == 0)
    def _flush():
      pltpu.sync_copy(table_sh, out_hbm.at[cid])

  return k(vals, idx, zeros_nd)


def _leaky(x):
  return jnp.where(x > 0, x, 0.2 * x)


def _elu(x):
  return jnp.where(x > 0, x, jnp.exp(jnp.minimum(x, 0.0)) - 1.0)


def _enc_body(x_ref, W0, b0, g0, be0, Wl, bl, Wr, br, xl_ref, xr_ref):
  h0 = jnp.dot(x_ref[...], W0[...], preferred_element_type=jnp.float32) + b0[...]
  m = jnp.mean(h0, axis=0, keepdims=True)
  v = jnp.mean((h0 - m) * (h0 - m), axis=0, keepdims=True)
  h = (h0 - m) / jnp.sqrt(v + 1e-5) * g0[...] + be0[...]
  h = jnp.maximum(h, 0.0)
  xl_ref[...] = jnp.dot(h, Wl[...], preferred_element_type=jnp.float32) + bl[...]
  xr_ref[...] = jnp.dot(h, Wr[...], preferred_element_type=jnp.float32) + br[...]


def _tc_encode(x, p):
  out = pl.pallas_call(
      _enc_body,
      out_shape=(jax.ShapeDtypeStruct((10000, 256), jnp.float32),
                 jax.ShapeDtypeStruct((10000, 256), jnp.float32)),
  )(x, p['W0'], p['b0'].reshape(1, -1), p['g0'].reshape(1, -1),
    p['be0'].reshape(1, -1), p['Wl1'], p['bl1'].reshape(1, -1),
    p['Wr1'], p['br1'].reshape(1, -1))
  return out


def _edge1_body(gxl_ref, gxr_ref, ea_ref, We, att, msg_ref, ex_ref, *, nvalid, blk):
  i = pl.program_id(0)
  gxl = gxl_ref[...]
  el = jnp.dot(ea_ref[...], We[...], preferred_element_type=jnp.float32)
  e = _leaky(gxl + gxr_ref[...] + el)
  rows = i * blk + lax.broadcasted_iota(jnp.int32, (blk, 1), 0)
  valid = rows < nvalid
  exs = []
  for h in range(4):
    eh = e[:, h * 64:(h + 1) * 64]
    ah = jnp.sum(eh * att[h:h + 1, :], axis=1, keepdims=True)
    exh = jnp.where(valid, jnp.exp(ah), 0.0)
    exs.append(exh)
    msg_ref[:, h * 64:(h + 1) * 64] = gxl[:, h * 64:(h + 1) * 64] * exh
  ex_ref[...] = jnp.concatenate(exs + [jnp.zeros((blk, 12), jnp.float32)], axis=1)


def _tc_edge1(gxl, gxr, ea_pad, p):
  blk = 1024
  grid = E_PAD // blk
  return pl.pallas_call(
      functools.partial(_edge1_body, nvalid=160000, blk=blk),
      grid=(grid,),
      in_specs=[
          pl.BlockSpec((blk, 256), lambda i: (i, 0)),
          pl.BlockSpec((blk, 256), lambda i: (i, 0)),
          pl.BlockSpec((blk, 16), lambda i: (i, 0)),
          pl.BlockSpec((16, 256), lambda i: (0, 0)),
          pl.BlockSpec((4, 64), lambda i: (0, 0)),
      ],
      out_specs=(pl.BlockSpec((blk, 256), lambda i: (i, 0)),
                 pl.BlockSpec((blk, 16), lambda i: (i, 0))),
      out_shape=(jax.ShapeDtypeStruct((E_PAD, 256), jnp.float32),
                 jax.ShapeDtypeStruct((E_PAD, 16), jnp.float32)),
  )(gxl, gxr, ea_pad, p['We1'], p['att1'])


def _mid_body(num_ref, den_ref, sm_ref, xl_ref, xr_ref, We, att, bias, h1_ref):
  sm = sm_ref[0] + sm_ref[1]
  cnt = jnp.maximum(sm[:, 16:17], 1.0)
  loop_attr = sm[:, 0:16] / cnt
  el = jnp.dot(loop_attr, We[...], preferred_element_type=jnp.float32)
  xl = xl_ref[...]
  e = _leaky(xl + xr_ref[...] + el)
  num = num_ref[0] + num_ref[1]
  den = den_ref[0] + den_ref[1]
  outs = []
  for h in range(4):
    eh = e[:, h * 64:(h + 1) * 64]
    ah = jnp.sum(eh * att[h:h + 1, :], axis=1, keepdims=True)
    exh = jnp.exp(ah)
    nh = num[:, h * 64:(h + 1) * 64] + xl[:, h * 64:(h + 1) * 64] * exh
    dh = jnp.maximum(den[:, h:h + 1] + exh, 1e-16)
    outs.append(nh / dh)
  h1_ref[...] = _elu(jnp.concatenate(outs, axis=1) + bias[...])


def _proj2_body(h1_ref, Wl2, bl2, Wr2, br2, x2_ref):
  h1 = h1_ref[...]
  xl2 = jnp.dot(h1, Wl2[...], preferred_element_type=jnp.float32) + bl2[...]
  xr2 = jnp.dot(h1, Wr2[...], preferred_element_type=jnp.float32) + br2[...]
  x2_ref[...] = jnp.concatenate([xl2, xr2], axis=1)


def _tc_mid(num2, den2, sm2, xl, xr, p):
  blk = 1000
  h1 = pl.pallas_call(
      _mid_body,
      grid=(10000 // blk,),
      in_specs=[
          pl.BlockSpec((2, blk, 256), lambda i: (0, i, 0)),
          pl.BlockSpec((2, blk, 16), lambda i: (0, i, 0)),
          pl.BlockSpec((2, blk, 32), lambda i: (0, i, 0)),
          pl.BlockSpec((blk, 256), lambda i: (i, 0)),
          pl.BlockSpec((blk, 256), lambda i: (i, 0)),
          pl.BlockSpec((16, 256), lambda i: (0, 0)),
          pl.BlockSpec((4, 64), lambda i: (0, 0)),
          pl.BlockSpec((1, 256), lambda i: (0, 0)),
      ],
      out_specs=pl.BlockSpec((blk, 256), lambda i: (i, 0)),
      out_shape=jax.ShapeDtypeStruct((10000, 256), jnp.float32),
  )(num2, den2, sm2, xl, xr, p['We1'], p['att1'], p['bias1'].reshape(1, -1))
  return pl.pallas_call(
      _proj2_body,
      out_shape=jax.ShapeDtypeStruct((10000, 128), jnp.float32),
  )(h1, p['Wl2'], p['bl2'].reshape(1, -1), p['Wr2'], p['br2'].reshape(1, -1))


def _edge2_body(gs_ref, gd_ref, ea_ref, We, att, feat_ref, *, nvalid, blk):
  i = pl.program_id(0)
  gxl = gs_ref[:, 0:64]
  el = jnp.dot(ea_ref[...], We[...], preferred_element_type=jnp.float32)
  e = _leaky(gxl + gd_ref[:, 64:128] + el)
  rows = i * blk + lax.broadcasted_iota(jnp.int32, (blk, 1), 0)
  valid = rows < nvalid
  a = jnp.sum(e * att[0:1, :], axis=1, keepdims=True)
  ex = jnp.where(valid, jnp.exp(a), 0.0)
  feat_ref[:, 0:64] = gxl * ex
  feat_ref[:, 64:80] = jnp.concatenate(
      [ex, jnp.zeros((blk, 15), jnp.float32)], axis=1)


def _tc_edge2(gs, gd, ea_pad, p):
  blk = 1024
  grid = E_PAD // blk
  return pl.pallas_call(
      functools.partial(_edge2_body, nvalid=160000, blk=blk),
      grid=(grid,),
      in_specs=[
          pl.BlockSpec((blk, 128), lambda i: (i, 0)),
          pl.BlockSpec((blk, 128), lambda i: (i, 0)),
          pl.BlockSpec((blk, 16), lambda i: (i, 0)),
          pl.BlockSpec((16, 64), lambda i: (0, 0)),
          pl.BlockSpec((1, 64), lambda i: (0, 0)),
      ],
      out_specs=pl.BlockSpec((blk, 80), lambda i: (i, 0)),
      out_shape=jax.ShapeDtypeStruct((E_PAD, 80), jnp.float32),
  )(gs, gd, ea_pad, p['We2'], p['att2'])


def _pool_body(scat_ref, x2_ref, att, bias, batch_ref, va_ref):
  xl = x2_ref[:, 0:64]
  e = _leaky(xl + x2_ref[:, 64:128])
  a = jnp.sum(e * att[0:1, :], axis=1, keepdims=True)
  ex = jnp.exp(a)
  scat = scat_ref[0] + scat_ref[1]
  num = scat[:, 0:64] + xl * ex
  den = jnp.maximum(scat[:, 64:65] + ex, 1e-16)
  h2 = _elu(num / den + bias[...])
  batch = batch_ref[...]
  gids = lax.broadcasted_iota(jnp.int32, (10000, 64), 1)
  oh = jnp.where(batch == gids, 1.0, 0.0)
  cnt = jnp.maximum(jnp.sum(oh, axis=0, keepdims=True), 1.0)
  mean = lax.dot_general(oh, h2, (((0,), (0,)), ((), ())),
                         preferred_element_type=jnp.float32) / cnt.T
  va_ref[:, 0:64] = mean

  def body(g, carry):
    mask = batch == g
    mg = jnp.max(jnp.where(mask, h2, -jnp.inf), axis=0, keepdims=True)
    mg = jnp.where(mg > -jnp.inf, mg, 0.0)
    va_ref[pl.ds(g, 1), 64:128] = mg
    return carry

  lax.fori_loop(0, 64, body, 0)


def _tc_pool(scat2, x2, batch, p):
  return pl.pallas_call(
      _pool_body,
      out_shape=jax.ShapeDtypeStruct((64, 128), jnp.float32),
  )(scat2, x2, p['att2'], p['bias2'].reshape(1, -1),
    batch.reshape(-1, 1))


def _head_body(va_ref, vb_ref, ea_ref, eb_ref, Wh1, bh1, gh, beh, Wh2, bh2,
               Wh3, bh3, Wp1, bp1, Wp2, bp2, Wg1, bg1, Wg2, bg2, out_ref):
  va = va_ref[...]
  vb = vb_ref[...]
  gat_c = jnp.concatenate([va + vb, jnp.abs(va - vb), va * vb], axis=1)
  ea = ea_ref[...]
  eb = eb_ref[...]
  enz_c = jnp.concatenate([ea + eb, jnp.abs(ea - eb), ea * eb], axis=1)
  h0 = jnp.dot(gat_c, Wh1[...], preferred_element_type=jnp.float32) + bh1[...]
  m = jnp.mean(h0, axis=0, keepdims=True)
  v = jnp.mean((h0 - m) * (h0 - m), axis=0, keepdims=True)
  h = jnp.maximum((h0 - m) / jnp.sqrt(v + 1e-5) * gh[...] + beh[...], 0.0)
  h = jnp.maximum(jnp.dot(h, Wh2[...], preferred_element_type=jnp.float32) + bh2[...], 0.0)
  gat_logits = jnp.dot(h, Wh3[...], preferred_element_type=jnp.float32) + bh3[...]
  hp = jnp.maximum(jnp.dot(enz_c, Wp1[...], preferred_element_type=jnp.float32) + bp1[...], 0.0)
  prior = jnp.dot(hp, Wp2[...], preferred_element_type=jnp.float32) + bp2[...]
  gate_in = jnp.concatenate([gat_c, enz_c], axis=1)
  hg = jnp.maximum(jnp.dot(gate_in, Wg1[...], preferred_element_type=jnp.float32) + bg1[...], 0.0)
  z = jnp.dot(hg, Wg2[...], preferred_element_type=jnp.float32) + bg2[...]
  alpha = 1.0 / (1.0 + jnp.exp(-z))
  out_ref[...] = alpha * gat_logits + (1.0 - alpha) * prior


def _tc_head(va, vb, enz_a, enz_b, p):
  return pl.pallas_call(
      _head_body,
      out_shape=jax.ShapeDtypeStruct((64, 1), jnp.float32),
  )(va, vb, enz_a, enz_b,
    p['Wh1'], p['bh1'].reshape(1, -1), p['gh'].reshape(1, -1),
    p['beh'].reshape(1, -1), p['Wh2'], p['bh2'].reshape(1, -1),
    p['Wh3'], p['bh3'].reshape(1, -1), p['Wp1'], p['bp1'].reshape(1, -1),
    p['Wp2'], p['bp2'].reshape(1, -1), p['Wg1'], p['bg1'].reshape(1, -1),
    p['Wg2'], p['bg2'].reshape(1, -1))


def _arm(x, edge_index, edge_attr, batch, p):
  E = edge_attr.shape[0]
  N = x.shape[0]
  pad = E_PAD - E
  src = jnp.concatenate([edge_index[0], jnp.zeros((pad,), jnp.int32)])
  dst = jnp.concatenate([edge_index[1], jnp.zeros((pad,), jnp.int32)])
  ea_pad = jnp.concatenate([edge_attr, jnp.zeros((pad, 16), jnp.float32)], axis=0)

  ones_col = jnp.concatenate([jnp.ones((E, 1), jnp.float32),
                              jnp.zeros((pad, 1), jnp.float32)], axis=0)
  sm_vals = jnp.concatenate(
      [ea_pad, ones_col, jnp.zeros((E_PAD, 15), jnp.float32)], axis=1)
  z32 = jnp.zeros((N, 32), jnp.float32)
  sm2 = _sc_scatter_add(sm_vals, dst, z32, N, 32)

  xl, xr = _tc_encode(x, p)
  gxl = _sc_gather(xl, src, 256)
  gxr = _sc_gather(xr, dst, 256)
  msg, ex = _tc_edge1(gxl, gxr, ea_pad, p)

  z128 = jnp.zeros((N, 128), jnp.float32)
  z16 = jnp.zeros((N, 16), jnp.float32)
  numA = _sc_scatter_add(msg[:, 0:128], dst, z128, N, 128)
  numB = _sc_scatter_add(msg[:, 128:256], dst, z128, N, 128)
  den = _sc_scatter_add(ex, dst, z16, N, 16)
  num2 = jnp.concatenate([numA, numB], axis=2)

  x2 = _tc_mid(num2, den, sm2, xl, xr, p)
  gs2 = _sc_gather(x2, src, 128)
  gd2 = _sc_gather(x2, dst, 128)
  feat2 = _tc_edge2(gs2, gd2, ea_pad, p)
  z80 = jnp.zeros((N, 80), jnp.float32)
  scat2 = _sc_scatter_add(feat2, dst, z80, N, 80)
  return _tc_pool(scat2, x2, batch, p)


def kernel(x_a, edge_index_a, edge_attr_a, batch_a, enz_a,
           x_b, edge_index_b, edge_attr_b, batch_b, enz_b, params):
  va = _arm(x_a, edge_index_a, edge_attr_a, batch_a, params)
  vb = _arm(x_b, edge_index_b, edge_attr_b, batch_b, params)
  return _tc_head(va, vb, enz_a, enz_b, params)


# speedup vs baseline: 11.4927x; 1.0697x over previous
import functools

import jax
import jax.numpy as jnp
from jax import lax
from jax.experimental import pallas as pl
from jax.experimental.pallas import tpu as pltpu
from jax.experimental.pallas import tpu_sc as plsc

NW = 32
CH = 128
E_PAD = 163840


def _sc_gather(table, idx, D):
  B = idx.shape[0]
  bpw = B // NW
  nch = bpw // CH
  mesh = plsc.VectorSubcoreMesh(core_axis_name="c", subcore_axis_name="s")

  @functools.partial(
      pl.kernel, mesh=mesh,
      out_type=jax.ShapeDtypeStruct((B, D), jnp.float32),
      scratch_types=[
          pltpu.VMEM((CH,), jnp.int32),
          pltpu.VMEM((CH,), jnp.int32),
          pltpu.VMEM((CH, D), jnp.float32),
          pltpu.VMEM((CH, D), jnp.float32),
          pltpu.SemaphoreType.DMA,
          pltpu.SemaphoreType.DMA,
      ],
  )
  def k(table_hbm, idx_hbm, out_hbm, idx0, idx1, rows0, rows1, sem0, sem1):
    wid = lax.axis_index("s") * 2 + lax.axis_index("c")

    def body(i, carry):
      b0 = wid * bpw + 2 * i * CH
      b1 = b0 + CH
      pltpu.sync_copy(idx_hbm.at[pl.ds(b0, CH)], idx0)
      pltpu.sync_copy(idx_hbm.at[pl.ds(b1, CH)], idx1)
      g0 = pltpu.async_copy(table_hbm.at[idx0], rows0, sem0)
      g1 = pltpu.async_copy(table_hbm.at[idx1], rows1, sem1)
      g0.wait()
      pltpu.sync_copy(rows0, out_hbm.at[pl.ds(b0, CH)])
      g1.wait()
      pltpu.sync_copy(rows1, out_hbm.at[pl.ds(b1, CH)])
      return carry

    lax.fori_loop(0, nch // 2, body, 0)

  return k(table, idx)


def _sc_scatter_add(vals, idx, zeros_nd, N, D):
  B = idx.shape[0]
  bpw = B // NW
  nch = bpw // CH
  mesh = plsc.VectorSubcoreMesh(core_axis_name="c", subcore_axis_name="s")

  @functools.partial(
      pl.kernel, mesh=mesh,
      out_type=jax.ShapeDtypeStruct((2, N, D), jnp.float32),
      scratch_types=[
          pltpu.VMEM((CH,), jnp.int32),
          pltpu.VMEM((CH,), jnp.int32),
          pltpu.VMEM((CH, D), jnp.float32),
          pltpu.VMEM((CH, D), jnp.float32),
          pltpu.VMEM_SHARED((N, D), jnp.float32),
          pltpu.SemaphoreType.DMA,
          pltpu.SemaphoreType.DMA,
      ],
  )
  def k(vals_hbm, idx_hbm, zero_hbm, out_hbm, idx0, idx1, vals0, vals1,
        table_sh, sem0, sem1):
    cid = lax.axis_index("c")
    sid = lax.axis_index("s")
    wid = sid * 2 + cid

    @pl.when(sid == 0)
    def _init():
      pltpu.sync_copy(zero_hbm, table_sh)

    plsc.subcore_barrier()

    def body(i, carry):
      b0 = wid * bpw + 2 * i * CH
      b1 = b0 + CH
      v0 = pltpu.async_copy(vals_hbm.at[pl.ds(b0, CH)], vals0, sem0)
      v1 = pltpu.async_copy(vals_hbm.at[pl.ds(b1, CH)], vals1, sem1)
      pltpu.sync_copy(idx_hbm.at[pl.ds(b0, CH)], idx0)
      pltpu.sync_copy(idx_hbm.at[pl.ds(b1, CH)], idx1)
      v0.wait()
      pltpu.sync_copy(vals0, table_sh.at[idx0], add=True)
      v1.wait()
      pltpu.sync_copy(vals1, table_sh.at[idx1], add=True)
      return carry

    lax.fori_loop(0, nch // 2, body, 0)
    plsc.subcore_barrier()

    @pl.when(sid == 0)
    def _flush():
      pltpu.sync_copy(table_sh, out_hbm.at[cid])

  return k(vals, idx, zeros_nd)


def _leaky(x):
  return jnp.where(x > 0, x, 0.2 * x)


def _elu(x):
  return jnp.where(x > 0, x, jnp.exp(jnp.minimum(x, 0.0)) - 1.0)


def _enc_body(x_ref, W0, b0, g0, be0, Wl, bl, Wr, br, xl_ref, xr_ref):
  h0 = jnp.dot(x_ref[...], W0[...], preferred_element_type=jnp.float32) + b0[...]
  m = jnp.mean(h0, axis=0, keepdims=True)
  v = jnp.mean((h0 - m) * (h0 - m), axis=0, keepdims=True)
  h = (h0 - m) / jnp.sqrt(v + 1e-5) * g0[...] + be0[...]
  h = jnp.maximum(h, 0.0)
  xl_ref[...] = jnp.dot(h, Wl[...], preferred_element_type=jnp.float32) + bl[...]
  xr_ref[...] = jnp.dot(h, Wr[...], preferred_element_type=jnp.float32) + br[...]


def _tc_encode(x, p):
  out = pl.pallas_call(
      _enc_body,
      out_shape=(jax.ShapeDtypeStruct((10000, 256), jnp.float32),
                 jax.ShapeDtypeStruct((10000, 256), jnp.float32)),
  )(x, p['W0'], p['b0'].reshape(1, -1), p['g0'].reshape(1, -1),
    p['be0'].reshape(1, -1), p['Wl1'], p['bl1'].reshape(1, -1),
    p['Wr1'], p['br1'].reshape(1, -1))
  return out


def _edge1_body(gxl_ref, gxr_ref, ea_ref, We, att, msg_ref, ex_ref, *, nvalid, blk):
  i = pl.program_id(0)
  gxl = gxl_ref[...]
  el = jnp.dot(ea_ref[...], We[...], preferred_element_type=jnp.float32)
  e = _leaky(gxl + gxr_ref[...] + el)
  rows = i * blk + lax.broadcasted_iota(jnp.int32, (blk, 1), 0)
  valid = rows < nvalid
  exs = []
  for h in range(4):
    eh = e[:, h * 64:(h + 1) * 64]
    ah = jnp.sum(eh * att[h:h + 1, :], axis=1, keepdims=True)
    exh = jnp.where(valid, jnp.exp(ah), 0.0)
    exs.append(exh)
    msg_ref[:, h * 64:(h + 1) * 64] = gxl[:, h * 64:(h + 1) * 64] * exh
  ex_ref[...] = jnp.concatenate(exs + [jnp.zeros((blk, 12), jnp.float32)], axis=1)


def _tc_edge1(gxl, gxr, ea_pad, p):
  blk = 1024
  grid = E_PAD // blk
  return pl.pallas_call(
      functools.partial(_edge1_body, nvalid=160000, blk=blk),
      grid=(grid,),
      in_specs=[
          pl.BlockSpec((blk, 256), lambda i: (i, 0)),
          pl.BlockSpec((blk, 256), lambda i: (i, 0)),
          pl.BlockSpec((blk, 16), lambda i: (i, 0)),
          pl.BlockSpec((16, 256), lambda i: (0, 0)),
          pl.BlockSpec((4, 64), lambda i: (0, 0)),
      ],
      out_specs=(pl.BlockSpec((blk, 256), lambda i: (i, 0)),
                 pl.BlockSpec((blk, 16), lambda i: (i, 0))),
      out_shape=(jax.ShapeDtypeStruct((E_PAD, 256), jnp.float32),
                 jax.ShapeDtypeStruct((E_PAD, 16), jnp.float32)),
  )(gxl, gxr, ea_pad, p['We1'], p['att1'])


def _mid_body(num_ref, den_ref, sm_ref, xl_ref, xr_ref, We, att, bias, h1_ref):
  sm = sm_ref[0] + sm_ref[1]
  cnt = jnp.maximum(sm[:, 16:17], 1.0)
  loop_attr = sm[:, 0:16] / cnt
  el = jnp.dot(loop_attr, We[...], preferred_element_type=jnp.float32)
  xl = xl_ref[...]
  e = _leaky(xl + xr_ref[...] + el)
  num = num_ref[0] + num_ref[1]
  den = den_ref[0] + den_ref[1]
  outs = []
  for h in range(4):
    eh = e[:, h * 64:(h + 1) * 64]
    ah = jnp.sum(eh * att[h:h + 1, :], axis=1, keepdims=True)
    exh = jnp.exp(ah)
    nh = num[:, h * 64:(h + 1) * 64] + xl[:, h * 64:(h + 1) * 64] * exh
    dh = jnp.maximum(den[:, h:h + 1] + exh, 1e-16)
    outs.append(nh / dh)
  h1_ref[...] = _elu(jnp.concatenate(outs, axis=1) + bias[...])


def _proj2_body(h1_ref, Wl2, bl2, Wr2, br2, x2_ref):
  h1 = h1_ref[...]
  xl2 = jnp.dot(h1, Wl2[...], preferred_element_type=jnp.float32) + bl2[...]
  xr2 = jnp.dot(h1, Wr2[...], preferred_element_type=jnp.float32) + br2[...]
  x2_ref[...] = jnp.concatenate([xl2, xr2], axis=1)


def _tc_mid(num2, den2, sm2, xl, xr, p):
  blk = 1000
  h1 = pl.pallas_call(
      _mid_body,
      grid=(10000 // blk,),
      in_specs=[
          pl.BlockSpec((2, blk, 256), lambda i: (0, i, 0)),
          pl.BlockSpec((2, blk, 16), lambda i: (0, i, 0)),
          pl.BlockSpec((2, blk, 32), lambda i: (0, i, 0)),
          pl.BlockSpec((blk, 256), lambda i: (i, 0)),
          pl.BlockSpec((blk, 256), lambda i: (i, 0)),
          pl.BlockSpec((16, 256), lambda i: (0, 0)),
          pl.BlockSpec((4, 64), lambda i: (0, 0)),
          pl.BlockSpec((1, 256), lambda i: (0, 0)),
      ],
      out_specs=pl.BlockSpec((blk, 256), lambda i: (i, 0)),
      out_shape=jax.ShapeDtypeStruct((10000, 256), jnp.float32),
  )(num2, den2, sm2, xl, xr, p['We1'], p['att1'], p['bias1'].reshape(1, -1))
  return pl.pallas_call(
      _proj2_body,
      out_shape=jax.ShapeDtypeStruct((10000, 128), jnp.float32),
  )(h1, p['Wl2'], p['bl2'].reshape(1, -1), p['Wr2'], p['br2'].reshape(1, -1))


def _edge2_body(gs_ref, gd_ref, ea_ref, We, att, feat_ref, *, nvalid, blk):
  i = pl.program_id(0)
  gxl = gs_ref[:, 0:64]
  el = jnp.dot(ea_ref[...], We[...], preferred_element_type=jnp.float32)
  e = _leaky(gxl + gd_ref[:, 64:128] + el)
  rows = i * blk + lax.broadcasted_iota(jnp.int32, (blk, 1), 0)
  valid = rows < nvalid
  a = jnp.sum(e * att[0:1, :], axis=1, keepdims=True)
  ex = jnp.where(valid, jnp.exp(a), 0.0)
  feat_ref[:, 0:64] = gxl * ex
  feat_ref[:, 64:80] = jnp.concatenate(
      [ex, jnp.zeros((blk, 15), jnp.float32)], axis=1)


def _tc_edge2(gs, gd, ea_pad, p):
  blk = 1024
  grid = E_PAD // blk
  return pl.pallas_call(
      functools.partial(_edge2_body, nvalid=160000, blk=blk),
      grid=(grid,),
      in_specs=[
          pl.BlockSpec((blk, 128), lambda i: (i, 0)),
          pl.BlockSpec((blk, 128), lambda i: (i, 0)),
          pl.BlockSpec((blk, 16), lambda i: (i, 0)),
          pl.BlockSpec((16, 64), lambda i: (0, 0)),
          pl.BlockSpec((1, 64), lambda i: (0, 0)),
      ],
      out_specs=pl.BlockSpec((blk, 80), lambda i: (i, 0)),
      out_shape=jax.ShapeDtypeStruct((E_PAD, 80), jnp.float32),
  )(gs, gd, ea_pad, p['We2'], p['att2'])


def _pool_body(scat_ref, x2_ref, att, bias, batch_ref, va_ref):
  xl = x2_ref[:, 0:64]
  e = _leaky(xl + x2_ref[:, 64:128])
  a = jnp.sum(e * att[0:1, :], axis=1, keepdims=True)
  ex = jnp.exp(a)
  scat = scat_ref[0] + scat_ref[1]
  num = scat[:, 0:64] + xl * ex
  den = jnp.maximum(scat[:, 64:65] + ex, 1e-16)
  h2 = _elu(num / den + bias[...])
  batch = batch_ref[...]
  gids = lax.broadcasted_iota(jnp.int32, (10000, 64), 1)
  oh = jnp.where(batch == gids, 1.0, 0.0)
  cnt = jnp.maximum(jnp.sum(oh, axis=0, keepdims=True), 1.0)
  mean = lax.dot_general(oh, h2, (((0,), (0,)), ((), ())),
                         preferred_element_type=jnp.float32) / cnt.T
  va_ref[:, 0:64] = mean

  def body(g, carry):
    mask = batch == g
    mg = jnp.max(jnp.where(mask, h2, -jnp.inf), axis=0, keepdims=True)
    mg = jnp.where(mg > -jnp.inf, mg, 0.0)
    va_ref[pl.ds(g, 1), 64:128] = mg
    return carry

  lax.fori_loop(0, 64, body, 0)


def _tc_pool(scat2, x2, batch, p):
  return pl.pallas_call(
      _pool_body,
      out_shape=jax.ShapeDtypeStruct((64, 128), jnp.float32),
  )(scat2, x2, p['att2'], p['bias2'].reshape(1, -1),
    batch.reshape(-1, 1))


def _head_body(va_ref, vb_ref, ea_ref, eb_ref, Wh1, bh1, gh, beh, Wh2, bh2,
               Wh3, bh3, Wp1, bp1, Wp2, bp2, Wg1, bg1, Wg2, bg2, out_ref):
  va = va_ref[...]
  vb = vb_ref[...]
  gat_c = jnp.concatenate([va + vb, jnp.abs(va - vb), va * vb], axis=1)
  ea = ea_ref[...]
  eb = eb_ref[...]
  enz_c = jnp.concatenate([ea + eb, jnp.abs(ea - eb), ea * eb], axis=1)
  h0 = jnp.dot(gat_c, Wh1[...], preferred_element_type=jnp.float32) + bh1[...]
  m = jnp.mean(h0, axis=0, keepdims=True)
  v = jnp.mean((h0 - m) * (h0 - m), axis=0, keepdims=True)
  h = jnp.maximum((h0 - m) / jnp.sqrt(v + 1e-5) * gh[...] + beh[...], 0.0)
  h = jnp.maximum(jnp.dot(h, Wh2[...], preferred_element_type=jnp.float32) + bh2[...], 0.0)
  gat_logits = jnp.dot(h, Wh3[...], preferred_element_type=jnp.float32) + bh3[...]
  hp = jnp.maximum(jnp.dot(enz_c, Wp1[...], preferred_element_type=jnp.float32) + bp1[...], 0.0)
  prior = jnp.dot(hp, Wp2[...], preferred_element_type=jnp.float32) + bp2[...]
  gate_in = jnp.concatenate([gat_c, enz_c], axis=1)
  hg = jnp.maximum(jnp.dot(gate_in, Wg1[...], preferred_element_type=jnp.float32) + bg1[...], 0.0)
  z = jnp.dot(hg, Wg2[...], preferred_element_type=jnp.float32) + bg2[...]
  alpha = 1.0 / (1.0 + jnp.exp(-z))
  out_ref[...] = alpha * gat_logits + (1.0 - alpha) * prior


def _tc_head(va, vb, enz_a, enz_b, p):
  return pl.pallas_call(
      _head_body,
      out_shape=jax.ShapeDtypeStruct((64, 1), jnp.float32),
  )(va, vb, enz_a, enz_b,
    p['Wh1'], p['bh1'].reshape(1, -1), p['gh'].reshape(1, -1),
    p['beh'].reshape(1, -1), p['Wh2'], p['bh2'].reshape(1, -1),
    p['Wh3'], p['bh3'].reshape(1, -1), p['Wp1'], p['bp1'].reshape(1, -1),
    p['Wp2'], p['bp2'].reshape(1, -1), p['Wg1'], p['bg1'].reshape(1, -1),
    p['Wg2'], p['bg2'].reshape(1, -1))


def _arm(x, edge_index, edge_attr, batch, p):
  E = edge_attr.shape[0]
  N = x.shape[0]
  pad = E_PAD - E
  src = jnp.concatenate([edge_index[0], jnp.zeros((pad,), jnp.int32)])
  dst = jnp.concatenate([edge_index[1], jnp.zeros((pad,), jnp.int32)])
  ea_pad = jnp.concatenate([edge_attr, jnp.zeros((pad, 16), jnp.float32)], axis=0)

  ones_col = jnp.concatenate([jnp.ones((E, 1), jnp.float32),
                              jnp.zeros((pad, 1), jnp.float32)], axis=0)
  sm_vals = jnp.concatenate(
      [ea_pad, ones_col, jnp.zeros((E_PAD, 15), jnp.float32)], axis=1)
  z32 = jnp.zeros((N, 32), jnp.float32)
  sm2 = _sc_scatter_add(sm_vals, dst, z32, N, 32)

  xl, xr = _tc_encode(x, p)
  gxl = _sc_gather(xl, src, 256)
  gxr = _sc_gather(xr, dst, 256)
  msg, ex = _tc_edge1(gxl, gxr, ea_pad, p)

  z128 = jnp.zeros((N, 128), jnp.float32)
  z16 = jnp.zeros((N, 16), jnp.float32)
  numA = _sc_scatter_add(msg[:, 0:128], dst, z128, N, 128)
  numB = _sc_scatter_add(msg[:, 128:256], dst, z128, N, 128)
  den = _sc_scatter_add(ex, dst, z16, N, 16)
  num2 = jnp.concatenate([numA, numB], axis=2)

  x2 = _tc_mid(num2, den, sm2, xl, xr, p)
  gs2 = _sc_gather(x2, src, 128)
  gd2 = _sc_gather(x2, dst, 128)
  feat2 = _tc_edge2(gs2, gd2, ea_pad, p)
  z80 = jnp.zeros((N, 80), jnp.float32)
  scat2 = _sc_scatter_add(feat2, dst, z80, N, 80)
  return _tc_pool(scat2, x2, batch, p)


def kernel(x_a, edge_index_a, edge_attr_a, batch_a, enz_a,
           x_b, edge_index_b, edge_attr_b, batch_b, enz_b, params):
  va = _arm(x_a, edge_index_a, edge_attr_a, batch_a, params)
  vb = _arm(x_b, edge_index_b, edge_attr_b, batch_b, params)
  return _tc_head(va, vb, enz_a, enz_b, params)


# speedup vs baseline: 11.5811x; 1.0077x over previous
import functools

import jax
import jax.numpy as jnp
from jax import lax
from jax.experimental import pallas as pl
from jax.experimental.pallas import tpu as pltpu
from jax.experimental.pallas import tpu_sc as plsc

NW = 32
CH = 128
E_PAD = 163840


def _sc_gather(table, idx, D):
  B = idx.shape[0]
  bpw = B // NW
  nch = bpw // CH
  NB = 2 if D > 128 else 4
  mesh = plsc.VectorSubcoreMesh(core_axis_name="c", subcore_axis_name="s")

  @functools.partial(
      pl.kernel, mesh=mesh,
      out_type=jax.ShapeDtypeStruct((B, D), jnp.float32),
      scratch_types=([pltpu.VMEM((CH,), jnp.int32)] * NB +
                     [pltpu.VMEM((CH, D), jnp.float32)] * NB +
                     [pltpu.SemaphoreType.DMA] * NB),
  )
  def k(table_hbm, idx_hbm, out_hbm, *scr):
    idxs = scr[:NB]
    rows = scr[NB:2 * NB]
    sems = scr[2 * NB:]
    wid = lax.axis_index("s") * 2 + lax.axis_index("c")

    def body(i, carry):
      base = wid * bpw + NB * i * CH
      for b in range(NB):
        pltpu.sync_copy(idx_hbm.at[pl.ds(base + b * CH, CH)], idxs[b])
      hs = [pltpu.async_copy(table_hbm.at[idxs[b]], rows[b], sems[b])
            for b in range(NB)]
      for b in range(NB):
        hs[b].wait()
        pltpu.sync_copy(rows[b], out_hbm.at[pl.ds(base + b * CH, CH)])
      return carry

    lax.fori_loop(0, nch // NB, body, 0)

  return k(table, idx)


def _sc_scatter_add(vals, idx, zeros_nd, N, D):
  B = idx.shape[0]
  bpw = B // NW
  nch = bpw // CH
  NB = 2
  mesh = plsc.VectorSubcoreMesh(core_axis_name="c", subcore_axis_name="s")

  @functools.partial(
      pl.kernel, mesh=mesh,
      out_type=jax.ShapeDtypeStruct((2, N, D), jnp.float32),
      scratch_types=([pltpu.VMEM((CH,), jnp.int32)] * NB +
                     [pltpu.VMEM((CH, D), jnp.float32)] * NB +
                     [pltpu.VMEM_SHARED((N, D), jnp.float32)] +
                     [pltpu.SemaphoreType.DMA] * NB),
  )
  def k(vals_hbm, idx_hbm, zero_hbm, out_hbm, *scr):
    idxs = scr[:NB]
    vals_v = scr[NB:2 * NB]
    table_sh = scr[2 * NB]
    sems = scr[2 * NB + 1:]
    cid = lax.axis_index("c")
    sid = lax.axis_index("s")
    wid = sid * 2 + cid

    @pl.when(sid == 0)
    def _init():
      pltpu.sync_copy(zero_hbm, table_sh)

    plsc.subcore_barrier()

    def body(i, carry):
      base = wid * bpw + NB * i * CH
      hs = [pltpu.async_copy(vals_hbm.at[pl.ds(base + b * CH, CH)],
                             vals_v[b], sems[b]) for b in range(NB)]
      for b in range(NB):
        pltpu.sync_copy(idx_hbm.at[pl.ds(base + b * CH, CH)], idxs[b])
      for b in range(NB):
        hs[b].wait()
        pltpu.sync_copy(vals_v[b], table_sh.at[idxs[b]], add=True)
      return carry

    lax.fori_loop(0, nch // NB, body, 0)
    plsc.subcore_barrier()

    @pl.when(sid == 0)
    def _flush():
      pltpu.sync_copy(table_sh, out_hbm.at[cid])

  return k(vals, idx, zeros_nd)


def _leaky(x):
  return jnp.where(x > 0, x, 0.2 * x)


def _elu(x):
  return jnp.where(x > 0, x, jnp.exp(jnp.minimum(x, 0.0)) - 1.0)


def _enc_body(x_ref, W0, b0, g0, be0, Wl, bl, Wr, br, xl_ref, xr_ref):
  h0 = jnp.dot(x_ref[...], W0[...], preferred_element_type=jnp.float32) + b0[...]
  m = jnp.mean(h0, axis=0, keepdims=True)
  v = jnp.mean((h0 - m) * (h0 - m), axis=0, keepdims=True)
  h = (h0 - m) / jnp.sqrt(v + 1e-5) * g0[...] + be0[...]
  h = jnp.maximum(h, 0.0)
  xl_ref[...] = jnp.dot(h, Wl[...], preferred_element_type=jnp.float32) + bl[...]
  xr_ref[...] = jnp.dot(h, Wr[...], preferred_element_type=jnp.float32) + br[...]


def _tc_encode(x, p):
  out = pl.pallas_call(
      _enc_body,
      out_shape=(jax.ShapeDtypeStruct((10000, 256), jnp.float32),
                 jax.ShapeDtypeStruct((10000, 256), jnp.float32)),
  )(x, p['W0'], p['b0'].reshape(1, -1), p['g0'].reshape(1, -1),
    p['be0'].reshape(1, -1), p['Wl1'], p['bl1'].reshape(1, -1),
    p['Wr1'], p['br1'].reshape(1, -1))
  return out


def _edge1_body(gxl_ref, gxr_ref, ea_ref, We, att, msg_ref, ex_ref, *, nvalid, blk):
  i = pl.program_id(0)
  gxl = gxl_ref[...]
  el = jnp.dot(ea_ref[...], We[...], preferred_element_type=jnp.float32)
  e = _leaky(gxl + gxr_ref[...] + el)
  rows = i * blk + lax.broadcasted_iota(jnp.int32, (blk, 1), 0)
  valid = rows < nvalid
  exs = []
  for h in range(4):
    eh = e[:, h * 64:(h + 1) * 64]
    ah = jnp.sum(eh * att[h:h + 1, :], axis=1, keepdims=True)
    exh = jnp.where(valid, jnp.exp(ah), 0.0)
    exs.append(exh)
    msg_ref[:, h * 64:(h + 1) * 64] = gxl[:, h * 64:(h + 1) * 64] * exh
  ex_ref[...] = jnp.concatenate(exs + [jnp.zeros((blk, 12), jnp.float32)], axis=1)


def _tc_edge1(gxl, gxr, ea_pad, p):
  blk = 1024
  grid = E_PAD // blk
  return pl.pallas_call(
      functools.partial(_edge1_body, nvalid=160000, blk=blk),
      grid=(grid,),
      in_specs=[
          pl.BlockSpec((blk, 256), lambda i: (i, 0)),
          pl.BlockSpec((blk, 256), lambda i: (i, 0)),
          pl.BlockSpec((blk, 16), lambda i: (i, 0)),
          pl.BlockSpec((16, 256), lambda i: (0, 0)),
          pl.BlockSpec((4, 64), lambda i: (0, 0)),
      ],
      out_specs=(pl.BlockSpec((blk, 256), lambda i: (i, 0)),
                 pl.BlockSpec((blk, 16), lambda i: (i, 0))),
      out_shape=(jax.ShapeDtypeStruct((E_PAD, 256), jnp.float32),
                 jax.ShapeDtypeStruct((E_PAD, 16), jnp.float32)),
  )(gxl, gxr, ea_pad, p['We1'], p['att1'])


def _mid_body(num_ref, den_ref, sm_ref, xl_ref, xr_ref, We, att, bias, h1_ref):
  sm = sm_ref[0] + sm_ref[1]
  cnt = jnp.maximum(sm[:, 16:17], 1.0)
  loop_attr = sm[:, 0:16] / cnt
  el = jnp.dot(loop_attr, We[...], preferred_element_type=jnp.float32)
  xl = xl_ref[...]
  e = _leaky(xl + xr_ref[...] + el)
  num = num_ref[0] + num_ref[1]
  den = den_ref[0] + den_ref[1]
  outs = []
  for h in range(4):
    eh = e[:, h * 64:(h + 1) * 64]
    ah = jnp.sum(eh * att[h:h + 1, :], axis=1, keepdims=True)
    exh = jnp.exp(ah)
    nh = num[:, h * 64:(h + 1) * 64] + xl[:, h * 64:(h + 1) * 64] * exh
    dh = jnp.maximum(den[:, h:h + 1] + exh, 1e-16)
    outs.append(nh / dh)
  h1_ref[...] = _elu(jnp.concatenate(outs, axis=1) + bias[...])


def _proj2_body(h1_ref, Wl2, bl2, Wr2, br2, x2_ref):
  h1 = h1_ref[...]
  xl2 = jnp.dot(h1, Wl2[...], preferred_element_type=jnp.float32) + bl2[...]
  xr2 = jnp.dot(h1, Wr2[...], preferred_element_type=jnp.float32) + br2[...]
  x2_ref[...] = jnp.concatenate([xl2, xr2], axis=1)


def _tc_mid(num2, den2, sm2, xl, xr, p):
  blk = 1000
  h1 = pl.pallas_call(
      _mid_body,
      grid=(10000 // blk,),
      in_specs=[
          pl.BlockSpec((2, blk, 256), lambda i: (0, i, 0)),
          pl.BlockSpec((2, blk, 16), lambda i: (0, i, 0)),
          pl.BlockSpec((2, blk, 32), lambda i: (0, i, 0)),
          pl.BlockSpec((blk, 256), lambda i: (i, 0)),
          pl.BlockSpec((blk, 256), lambda i: (i, 0)),
          pl.BlockSpec((16, 256), lambda i: (0, 0)),
          pl.BlockSpec((4, 64), lambda i: (0, 0)),
          pl.BlockSpec((1, 256), lambda i: (0, 0)),
      ],
      out_specs=pl.BlockSpec((blk, 256), lambda i: (i, 0)),
      out_shape=jax.ShapeDtypeStruct((10000, 256), jnp.float32),
  )(num2, den2, sm2, xl, xr, p['We1'], p['att1'], p['bias1'].reshape(1, -1))
  return pl.pallas_call(
      _proj2_body,
      out_shape=jax.ShapeDtypeStruct((10000, 128), jnp.float32),
  )(h1, p['Wl2'], p['bl2'].reshape(1, -1), p['Wr2'], p['br2'].reshape(1, -1))


def _edge2_body(gs_ref, gd_ref, ea_ref, We, att, feat_ref, *, nvalid, blk):
  i = pl.program_id(0)
  gxl = gs_ref[:, 0:64]
  el = jnp.dot(ea_ref[...], We[...], preferred_element_type=jnp.float32)
  e = _leaky(gxl + gd_ref[:, 64:128] + el)
  rows = i * blk + lax.broadcasted_iota(jnp.int32, (blk, 1), 0)
  valid = rows < nvalid
  a = jnp.sum(e * att[0:1, :], axis=1, keepdims=True)
  ex = jnp.where(valid, jnp.exp(a), 0.0)
  feat_ref[:, 0:64] = gxl * ex
  feat_ref[:, 64:80] = jnp.concatenate(
      [ex, jnp.zeros((blk, 15), jnp.float32)], axis=1)


def _tc_edge2(gs, gd, ea_pad, p):
  blk = 1024
  grid = E_PAD // blk
  return pl.pallas_call(
      functools.partial(_edge2_body, nvalid=160000, blk=blk),
      grid=(grid,),
      in_specs=[
          pl.BlockSpec((blk, 128), lambda i: (i, 0)),
          pl.BlockSpec((blk, 128), lambda i: (i, 0)),
          pl.BlockSpec((blk, 16), lambda i: (i, 0)),
          pl.BlockSpec((16, 64), lambda i: (0, 0)),
          pl.BlockSpec((1, 64), lambda i: (0, 0)),
      ],
      out_specs=pl.BlockSpec((blk, 80), lambda i: (i, 0)),
      out_shape=jax.ShapeDtypeStruct((E_PAD, 80), jnp.float32),
  )(gs, gd, ea_pad, p['We2'], p['att2'])


def _pool_body(scat_ref, x2_ref, att, bias, batch_ref, va_ref):
  xl = x2_ref[:, 0:64]
  e = _leaky(xl + x2_ref[:, 64:128])
  a = jnp.sum(e * att[0:1, :], axis=1, keepdims=True)
  ex = jnp.exp(a)
  scat = scat_ref[0] + scat_ref[1]
  num = scat[:, 0:64] + xl * ex
  den = jnp.maximum(scat[:, 64:65] + ex, 1e-16)
  h2 = _elu(num / den + bias[...])
  batch = batch_ref[...]
  gids = lax.broadcasted_iota(jnp.int32, (10000, 64), 1)
  oh = jnp.where(batch == gids, 1.0, 0.0)
  cnt = jnp.maximum(jnp.sum(oh, axis=0, keepdims=True), 1.0)
  mean = lax.dot_general(oh, h2, (((0,), (0,)), ((), ())),
                         preferred_element_type=jnp.float32) / cnt.T
  va_ref[:, 0:64] = mean

  def body(g, carry):
    mask = batch == g
    mg = jnp.max(jnp.where(mask, h2, -jnp.inf), axis=0, keepdims=True)
    mg = jnp.where(mg > -jnp.inf, mg, 0.0)
    va_ref[pl.ds(g, 1), 64:128] = mg
    return carry

  lax.fori_loop(0, 64, body, 0)


def _tc_pool(scat2, x2, batch, p):
  return pl.pallas_call(
      _pool_body,
      out_shape=jax.ShapeDtypeStruct((64, 128), jnp.float32),
  )(scat2, x2, p['att2'], p['bias2'].reshape(1, -1),
    batch.reshape(-1, 1))


def _head_body(va_ref, vb_ref, ea_ref, eb_ref, Wh1, bh1, gh, beh, Wh2, bh2,
               Wh3, bh3, Wp1, bp1, Wp2, bp2, Wg1, bg1, Wg2, bg2, out_ref):
  va = va_ref[...]
  vb = vb_ref[...]
  gat_c = jnp.concatenate([va + vb, jnp.abs(va - vb), va * vb], axis=1)
  ea = ea_ref[...]
  eb = eb_ref[...]
  enz_c = jnp.concatenate([ea + eb, jnp.abs(ea - eb), ea * eb], axis=1)
  h0 = jnp.dot(gat_c, Wh1[...], preferred_element_type=jnp.float32) + bh1[...]
  m = jnp.mean(h0, axis=0, keepdims=True)
  v = jnp.mean((h0 - m) * (h0 - m), axis=0, keepdims=True)
  h = jnp.maximum((h0 - m) / jnp.sqrt(v + 1e-5) * gh[...] + beh[...], 0.0)
  h = jnp.maximum(jnp.dot(h, Wh2[...], preferred_element_type=jnp.float32) + bh2[...], 0.0)
  gat_logits = jnp.dot(h, Wh3[...], preferred_element_type=jnp.float32) + bh3[...]
  hp = jnp.maximum(jnp.dot(enz_c, Wp1[...], preferred_element_type=jnp.float32) + bp1[...], 0.0)
  prior = jnp.dot(hp, Wp2[...], preferred_element_type=jnp.float32) + bp2[...]
  gate_in = jnp.concatenate([gat_c, enz_c], axis=1)
  hg = jnp.maximum(jnp.dot(gate_in, Wg1[...], preferred_element_type=jnp.float32) + bg1[...], 0.0)
  z = jnp.dot(hg, Wg2[...], preferred_element_type=jnp.float32) + bg2[...]
  alpha = 1.0 / (1.0 + jnp.exp(-z))
  out_ref[...] = alpha * gat_logits + (1.0 - alpha) * prior


def _tc_head(va, vb, enz_a, enz_b, p):
  return pl.pallas_call(
      _head_body,
      out_shape=jax.ShapeDtypeStruct((64, 1), jnp.float32),
  )(va, vb, enz_a, enz_b,
    p['Wh1'], p['bh1'].reshape(1, -1), p['gh'].reshape(1, -1),
    p['beh'].reshape(1, -1), p['Wh2'], p['bh2'].reshape(1, -1),
    p['Wh3'], p['bh3'].reshape(1, -1), p['Wp1'], p['bp1'].reshape(1, -1),
    p['Wp2'], p['bp2'].reshape(1, -1), p['Wg1'], p['bg1'].reshape(1, -1),
    p['Wg2'], p['bg2'].reshape(1, -1))


def _arm(x, edge_index, edge_attr, batch, p):
  E = edge_attr.shape[0]
  N = x.shape[0]
  pad = E_PAD - E
  src = jnp.concatenate([edge_index[0], jnp.zeros((pad,), jnp.int32)])
  dst = jnp.concatenate([edge_index[1], jnp.zeros((pad,), jnp.int32)])
  ea_pad = jnp.concatenate([edge_attr, jnp.zeros((pad, 16), jnp.float32)], axis=0)

  ones_col = jnp.concatenate([jnp.ones((E, 1), jnp.float32),
                              jnp.zeros((pad, 1), jnp.float32)], axis=0)
  sm_vals = jnp.concatenate(
      [ea_pad, ones_col, jnp.zeros((E_PAD, 15), jnp.float32)], axis=1)
  z32 = jnp.zeros((N, 32), jnp.float32)
  sm2 = _sc_scatter_add(sm_vals, dst, z32, N, 32)

  xl, xr = _tc_encode(x, p)
  gxl = _sc_gather(xl, src, 256)
  gxr = _sc_gather(xr, dst, 256)
  msg, ex = _tc_edge1(gxl, gxr, ea_pad, p)

  z128 = jnp.zeros((N, 128), jnp.float32)
  z16 = jnp.zeros((N, 16), jnp.float32)
  numA = _sc_scatter_add(msg[:, 0:128], dst, z128, N, 128)
  numB = _sc_scatter_add(msg[:, 128:256], dst, z128, N, 128)
  den = _sc_scatter_add(ex, dst, z16, N, 16)
  num2 = jnp.concatenate([numA, numB], axis=2)

  x2 = _tc_mid(num2, den, sm2, xl, xr, p)
  gs2 = _sc_gather(x2, src, 128)
  gd2 = _sc_gather(x2, dst, 128)
  feat2 = _tc_edge2(gs2, gd2, ea_pad, p)
  z80 = jnp.zeros((N, 80), jnp.float32)
  scat2 = _sc_scatter_add(feat2, dst, z80, N, 80)
  return _tc_pool(scat2, x2, batch, p)


def kernel(x_a, edge_index_a, edge_attr_a, batch_a, enz_a,
           x_b, edge_index_b, edge_attr_b, batch_b, enz_b, params):
  va = _arm(x_a, edge_index_a, edge_attr_a, batch_a, params)
  vb = _arm(x_b, edge_index_b, edge_attr_b, batch_b, params)
  return _tc_head(va, vb, enz_a, enz_b, params)
